# baseline (speedup 1.0000x reference)
import jax
import jax.numpy as jnp
from jax.experimental import pallas as pl
from jax.experimental.pallas import tpu as pltpu

N_ATOMS = 16384
N_STRUCT = 16
N_ELEM = 95
EMB = 64
HID = 256
XDIM = 128

BLOCK = 4096
HALF = 2048


def _fused_body(coord_ref, elems_ref, ind_ref, epad_ref, w1_ref, b1_ref,
                w2_ref, b2_ref, w3_ref, b3_ref, out_ref,
                wpad_ref, acc_ref, cnt_ref):
    i = pl.program_id(0)

    @pl.when(i == 0)
    def _init():
        wpad_ref[...] = jnp.dot(epad_ref[...], w1_ref[...],
                                preferred_element_type=jnp.float32, precision=jax.lax.Precision.HIGHEST)
        acc_ref[...] = jnp.zeros_like(acc_ref)
        cnt_ref[...] = jnp.zeros_like(cnt_ref)

    wpad = wpad_ref[...]
    w2 = w2_ref[...]
    b1 = b1_ref[0, :]
    b2 = b2_ref[0, :]

    for h in range(BLOCK // HALF):
        sl = pl.ds(h * HALF, HALF)
        elems = elems_ref[0, 0, sl]
        onehot = (jax.lax.broadcasted_iota(jnp.int32, (HALF, N_ELEM), 1)
                  == elems[:, None]).astype(jnp.float32)
        x = jnp.concatenate(
            [onehot, coord_ref[sl, :],
             jnp.zeros((HALF, XDIM - N_ELEM - 3), jnp.float32)], axis=1)
        hid = jnp.tanh(jnp.dot(x, wpad, preferred_element_type=jnp.float32, precision=jax.lax.Precision.HIGHEST)
                       + b1)
        hid = jnp.tanh(jnp.dot(hid, w2, preferred_element_type=jnp.float32, precision=jax.lax.Precision.HIGHEST)
                       + b2)
        ind = ind_ref[0, 0, sl]
        seg_t = (jax.lax.broadcasted_iota(jnp.int32, (N_STRUCT, HALF), 0)
                 == ind[None, :]).astype(jnp.float32)
        acc_ref[...] += jnp.dot(seg_t, hid,
                                preferred_element_type=jnp.float32, precision=jax.lax.Precision.HIGHEST)
        cnt_ref[...] += jnp.sum(seg_t, axis=1, keepdims=True)

    @pl.when(i == pl.num_programs(0) - 1)
    def _fin():
        out_ref[...] = (jnp.dot(acc_ref[...], w3_ref[...],
                                preferred_element_type=jnp.float32, precision=jax.lax.Precision.HIGHEST)
                        + b3_ref[0, 0] * cnt_ref[...])


@jax.jit
def kernel(coord, elems, ind_1, elem_embed, W1, b1, W2, b2, W3, b3):
    n = coord.shape[0]
    grid = n // BLOCK
    elems3 = elems.astype(jnp.int32).reshape(grid, 1, BLOCK)
    ind3 = ind_1.astype(jnp.int32).reshape(grid, 1, BLOCK)
    epad = jnp.concatenate([
        jnp.concatenate([elem_embed,
                         jnp.zeros((N_ELEM, 3), jnp.float32)], axis=1),
        jnp.concatenate([jnp.zeros((3, EMB), jnp.float32),
                         jnp.eye(3, dtype=jnp.float32)], axis=1),
        jnp.zeros((XDIM - N_ELEM - 3, EMB + 3), jnp.float32),
    ], axis=0)

    out = pl.pallas_call(
        _fused_body,
        grid=(grid,),
        in_specs=[
            pl.BlockSpec((BLOCK, 3), lambda i: (i, 0)),
            pl.BlockSpec((1, 1, BLOCK), lambda i: (i, 0, 0)),
            pl.BlockSpec((1, 1, BLOCK), lambda i: (i, 0, 0)),
            pl.BlockSpec((XDIM, EMB + 3), lambda i: (0, 0)),
            pl.BlockSpec((EMB + 3, HID), lambda i: (0, 0)),
            pl.BlockSpec((1, HID), lambda i: (0, 0)),
            pl.BlockSpec((HID, HID), lambda i: (0, 0)),
            pl.BlockSpec((1, HID), lambda i: (0, 0)),
            pl.BlockSpec((HID, 1), lambda i: (0, 0)),
            pl.BlockSpec((1, 1), lambda i: (0, 0)),
        ],
        out_specs=pl.BlockSpec((N_STRUCT, 1), lambda i: (0, 0)),
        out_shape=jax.ShapeDtypeStruct((N_STRUCT, 1), jnp.float32),
        scratch_shapes=[
            pltpu.VMEM((XDIM, HID), jnp.float32),
            pltpu.VMEM((N_STRUCT, HID), jnp.float32),
            pltpu.VMEM((N_STRUCT, 1), jnp.float32),
        ],
    )(coord, elems3, ind3, epad, W1, b1.reshape(1, HID), W2,
      b2.reshape(1, HID), W3, b3.reshape(1, 1))
    return out[:, 0]


# speedup vs baseline: 1.7524x; 1.7524x over previous
import jax
import jax.numpy as jnp
from jax.experimental import pallas as pl
from jax.experimental.pallas import tpu as pltpu

N_ATOMS = 16384
N_STRUCT = 16
N_ELEM = 95
EMB = 64
HID = 256
XDIM = 128

BLOCK = 4096
HALF = 2048

_F32 = jnp.float32
_BF16 = jnp.bfloat16


def _hi_lo(a):
    hi = a.astype(_BF16)
    lo = (a - hi.astype(_F32)).astype(_BF16)
    return hi, lo


def _fused_body(coord_ref, elems_ref, ind_ref, epad_ref, w1_ref, b1_ref,
                w2_ref, b2_ref, w3_ref, b3_ref, out_ref,
                wxh_ref, wxl_ref, w2h_ref, w2l_ref, acc_ref, cnt_ref):
    i = pl.program_id(0)

    @pl.when(i == 0)
    def _init():
        wext = jnp.dot(epad_ref[...], w1_ref[...],
                       preferred_element_type=_F32,
                       precision=jax.lax.Precision.HIGHEST)
        wh, wl = _hi_lo(wext)
        wxh_ref[...] = wh
        wxl_ref[...] = wl
        w2h, w2l = _hi_lo(w2_ref[...])
        w2h_ref[...] = w2h
        w2l_ref[...] = w2l
        acc_ref[...] = jnp.zeros_like(acc_ref)
        cnt_ref[...] = jnp.zeros_like(cnt_ref)

    wxh = wxh_ref[...]
    wxl = wxl_ref[...]
    w2h = w2h_ref[...]
    w2l = w2l_ref[...]
    b1 = b1_ref[0, :]
    b2 = b2_ref[0, :]

    for h in range(BLOCK // HALF):
        sl = pl.ds(h * HALF, HALF)
        elems = elems_ref[0, 0, sl]
        onehot = (jax.lax.broadcasted_iota(jnp.int32, (HALF, N_ELEM), 1)
                  == elems[:, None]).astype(_BF16)
        ch, cl = _hi_lo(coord_ref[sl, :])
        x = jnp.concatenate(
            [onehot, ch, cl,
             jnp.zeros((HALF, XDIM - N_ELEM - 6), _BF16)], axis=1)
        z1 = (jnp.dot(x, wxh, preferred_element_type=_F32)
              + jnp.dot(x, wxl, preferred_element_type=_F32) + b1)
        h1 = jnp.tanh(z1)
        hh, hl = _hi_lo(h1)
        z2 = (jnp.dot(hh, w2h, preferred_element_type=_F32)
              + (jnp.dot(hh, w2l, preferred_element_type=_F32)
                 + jnp.dot(hl, w2h, preferred_element_type=_F32)) + b2)
        h2 = jnp.tanh(z2)
        gh, gl = _hi_lo(h2)
        ind = ind_ref[0, 0, sl]
        seg_t = (jax.lax.broadcasted_iota(jnp.int32, (N_STRUCT, HALF), 0)
                 == ind[None, :]).astype(_BF16)
        acc_ref[...] += (jnp.dot(seg_t, gh, preferred_element_type=_F32)
                         + jnp.dot(seg_t, gl, preferred_element_type=_F32))
        cnt_ref[...] += jnp.sum(seg_t.astype(_F32), axis=1, keepdims=True)

    @pl.when(i == pl.num_programs(0) - 1)
    def _fin():
        out_ref[...] = (jnp.dot(acc_ref[...], w3_ref[...],
                                preferred_element_type=_F32,
                                precision=jax.lax.Precision.HIGHEST)
                        + b3_ref[0, 0] * cnt_ref[...])


@jax.jit
def kernel(coord, elems, ind_1, elem_embed, W1, b1, W2, b2, W3, b3):
    n = coord.shape[0]
    grid = n // BLOCK
    elems3 = elems.astype(jnp.int32).reshape(grid, 1, BLOCK)
    ind3 = ind_1.astype(jnp.int32).reshape(grid, 1, BLOCK)
    eye3 = jnp.concatenate([jnp.zeros((3, EMB), _F32),
                            jnp.eye(3, dtype=_F32)], axis=1)
    epad = jnp.concatenate([
        jnp.concatenate([elem_embed, jnp.zeros((N_ELEM, 3), _F32)], axis=1),
        eye3,
        eye3,
        jnp.zeros((XDIM - N_ELEM - 6, EMB + 3), _F32),
    ], axis=0)

    out = pl.pallas_call(
        _fused_body,
        grid=(grid,),
        in_specs=[
            pl.BlockSpec((BLOCK, 3), lambda i: (i, 0)),
            pl.BlockSpec((1, 1, BLOCK), lambda i: (i, 0, 0)),
            pl.BlockSpec((1, 1, BLOCK), lambda i: (i, 0, 0)),
            pl.BlockSpec((XDIM, EMB + 3), lambda i: (0, 0)),
            pl.BlockSpec((EMB + 3, HID), lambda i: (0, 0)),
            pl.BlockSpec((1, HID), lambda i: (0, 0)),
            pl.BlockSpec((HID, HID), lambda i: (0, 0)),
            pl.BlockSpec((1, HID), lambda i: (0, 0)),
            pl.BlockSpec((HID, 1), lambda i: (0, 0)),
            pl.BlockSpec((1, 1), lambda i: (0, 0)),
        ],
        out_specs=pl.BlockSpec((N_STRUCT, 1), lambda i: (0, 0)),
        out_shape=jax.ShapeDtypeStruct((N_STRUCT, 1), jnp.float32),
        scratch_shapes=[
            pltpu.VMEM((XDIM, HID), _BF16),
            pltpu.VMEM((XDIM, HID), _BF16),
            pltpu.VMEM((HID, HID), _BF16),
            pltpu.VMEM((HID, HID), _BF16),
            pltpu.VMEM((N_STRUCT, HID), _F32),
            pltpu.VMEM((N_STRUCT, 1), _F32),
        ],
    )(coord, elems3, ind3, epad, W1, b1.reshape(1, HID), W2,
      b2.reshape(1, HID), W3, b3.reshape(1, 1))
    return out[:, 0]


# speedup vs baseline: 2.5574x; 1.4593x over previous
import jax
import jax.numpy as jnp
from jax.experimental import pallas as pl
from jax.experimental.pallas import tpu as pltpu

N_ATOMS = 16384
N_STRUCT = 16
N_ELEM = 95
EMB = 64
HID = 256
XDIM = 128

BLOCK = 4096
HALF = 2048

_F32 = jnp.float32
_BF16 = jnp.bfloat16


def _hi_lo(a):
    hi = a.astype(_BF16)
    lo = (a - hi.astype(_F32)).astype(_BF16)
    return hi, lo


def _fused_body(coord_ref, elems_ref, ind_ref, emb_ref, w1c_ref, b1_ref,
                w2_ref, b2_ref, w3_ref, b3_ref, out_ref,
                wa_ref, wb_ref):
    i = pl.program_id(0)

    @pl.when(i == 0)
    def _init():
        m = jnp.dot(emb_ref[...], w1c_ref[0:EMB, :],
                    preferred_element_type=_F32)
        mh, ml = _hi_lo(m)
        zpad = jnp.zeros((XDIM - N_ELEM - 3, HID), _BF16)
        wa_ref[...] = jnp.concatenate(
            [mh, w1c_ref[EMB:, :].astype(_BF16), zpad], axis=0)
        wb_ref[...] = jnp.concatenate(
            [ml, jnp.zeros((3, HID), _BF16), zpad], axis=0)

    wa = wa_ref[...]
    wb = wb_ref[...]
    b1 = b1_ref[0, :]
    b2 = b2_ref[0, :]
    w2 = w2_ref[...]
    w3 = w3_ref[...]

    parts = []
    for h in range(BLOCK // HALF):
        sl = pl.ds(h * HALF, HALF)
        elems = elems_ref[0, 0, sl]
        onehot = (jax.lax.broadcasted_iota(jnp.int32, (HALF, N_ELEM), 1)
                  == elems[:, None]).astype(_BF16)
        x = jnp.concatenate(
            [onehot, coord_ref[sl, :].astype(_BF16),
             jnp.zeros((HALF, XDIM - N_ELEM - 3), _BF16)], axis=1)
        h1 = jnp.tanh(jnp.dot(x, wa, preferred_element_type=_F32)
                      + jnp.dot(x, wb, preferred_element_type=_F32) + b1)
        h2 = jnp.tanh(jnp.dot(h1, w2, preferred_element_type=_F32) + b2)
        per_atom = jnp.dot(h2, w3, preferred_element_type=_F32) + b3_ref[0, 0]
        ind = ind_ref[0, 0, sl]
        seg = (jax.lax.broadcasted_iota(jnp.int32, (HALF, N_STRUCT), 1)
               == ind[:, None]).astype(_F32)
        parts.append(jnp.dot(per_atom.reshape(1, HALF), seg,
                             preferred_element_type=_F32))
    part = parts[0] + parts[1]

    @pl.when(i == 0)
    def _first():
        out_ref[...] = part

    @pl.when(i != 0)
    def _acc():
        out_ref[...] += part


@jax.jit
def kernel(coord, elems, ind_1, elem_embed, W1, b1, W2, b2, W3, b3):
    n = coord.shape[0]
    grid = n // BLOCK
    elems3 = elems.astype(jnp.int32).reshape(grid, 1, BLOCK)
    ind3 = ind_1.astype(jnp.int32).reshape(grid, 1, BLOCK)

    out = pl.pallas_call(
        _fused_body,
        grid=(grid,),
        in_specs=[
            pl.BlockSpec((BLOCK, 3), lambda i: (i, 0)),
            pl.BlockSpec((1, 1, BLOCK), lambda i: (i, 0, 0)),
            pl.BlockSpec((1, 1, BLOCK), lambda i: (i, 0, 0)),
            pl.BlockSpec((N_ELEM, EMB), lambda i: (0, 0)),
            pl.BlockSpec((EMB + 3, HID), lambda i: (0, 0)),
            pl.BlockSpec((1, HID), lambda i: (0, 0)),
            pl.BlockSpec((HID, HID), lambda i: (0, 0)),
            pl.BlockSpec((1, HID), lambda i: (0, 0)),
            pl.BlockSpec((HID, 1), lambda i: (0, 0)),
            pl.BlockSpec((1, 1), lambda i: (0, 0)),
        ],
        out_specs=pl.BlockSpec((1, N_STRUCT), lambda i: (0, 0)),
        out_shape=jax.ShapeDtypeStruct((1, N_STRUCT), jnp.float32),
        scratch_shapes=[
            pltpu.VMEM((XDIM, HID), _BF16),
            pltpu.VMEM((XDIM, HID), _BF16),
        ],
    )(coord, elems3, ind3, elem_embed, W1, b1.reshape(1, HID), W2,
      b2.reshape(1, HID), W3, b3.reshape(1, 1))
    return out[0]


# speedup vs baseline: 2.6521x; 1.0370x over previous
import jax
import jax.numpy as jnp
from jax.experimental import pallas as pl
from jax.experimental.pallas import tpu as pltpu

N_ATOMS = 16384
N_STRUCT = 16
N_ELEM = 95
EMB = 64
HID = 256
XDIM = 128

BLOCK = 4096
HALF = 2048

_F32 = jnp.float32
_BF16 = jnp.bfloat16


def _hi_lo(a):
    hi = a.astype(_BF16)
    lo = (a - hi.astype(_F32)).astype(_BF16)
    return hi, lo


def _fused_body(coord_ref, elems_ref, ind_ref, emb_ref, w1c_ref, b1_ref,
                w2_ref, b2_ref, w3_ref, b3_ref, out_ref,
                wa_ref, wb_ref):
    i = pl.program_id(0)

    @pl.when(i == 0)
    def _init():
        m = jnp.dot(emb_ref[...], w1c_ref[0:EMB, :],
                    preferred_element_type=_F32)
        mh, ml = _hi_lo(m)
        zpad = jnp.zeros((XDIM - N_ELEM - 3, HID), _BF16)
        wa_ref[...] = jnp.concatenate(
            [mh, w1c_ref[EMB:, :].astype(_BF16), zpad], axis=0)
        wb_ref[...] = jnp.concatenate(
            [ml, jnp.zeros((3, HID), _BF16), zpad], axis=0)

    wa = wa_ref[...]
    wb = wb_ref[...]
    b1 = b1_ref[0, :]
    b2 = b2_ref[0, :]
    w2 = w2_ref[...]
    w3t = w3_ref[...]
    b3 = b3_ref[0, 0]

    parts = []
    for h in range(BLOCK // HALF):
        sl = pl.ds(h * HALF, HALF)
        elems = elems_ref[0, 0, sl]
        onehot = (jax.lax.broadcasted_iota(jnp.int32, (HALF, N_ELEM), 1)
                  == elems[:, None]).astype(_F32)
        x = jnp.concatenate(
            [onehot, coord_ref[sl, :],
             jnp.zeros((HALF, XDIM - N_ELEM - 3), _F32)], axis=1)
        xb = x.astype(_BF16)
        h1 = jnp.tanh(jnp.dot(xb, wa, preferred_element_type=_F32)
                      + jnp.dot(xb, wb, preferred_element_type=_F32) + b1)
        h2 = jnp.tanh(jnp.dot(h1, w2, preferred_element_type=_F32) + b2)
        pa16 = jnp.dot(h2, w3t, preferred_element_type=_F32) + b3
        ind = ind_ref[0, 0, sl]
        seg = (jax.lax.broadcasted_iota(jnp.int32, (HALF, N_STRUCT), 1)
               == ind[:, None]).astype(_F32)
        parts.append(jnp.sum(seg * pa16, axis=0)[None, :])
    part = parts[0] + parts[1]

    @pl.when(i == 0)
    def _first():
        out_ref[...] = part

    @pl.when(i != 0)
    def _acc():
        out_ref[...] += part


@jax.jit
def kernel(coord, elems, ind_1, elem_embed, W1, b1, W2, b2, W3, b3):
    n = coord.shape[0]
    grid = n // BLOCK
    elems3 = elems.astype(jnp.int32).reshape(grid, 1, BLOCK)
    ind3 = ind_1.astype(jnp.int32).reshape(grid, 1, BLOCK)
    w3t = jnp.tile(W3, (1, N_STRUCT))

    out = pl.pallas_call(
        _fused_body,
        grid=(grid,),
        in_specs=[
            pl.BlockSpec((BLOCK, 3), lambda i: (i, 0)),
            pl.BlockSpec((1, 1, BLOCK), lambda i: (i, 0, 0)),
            pl.BlockSpec((1, 1, BLOCK), lambda i: (i, 0, 0)),
            pl.BlockSpec((N_ELEM, EMB), lambda i: (0, 0)),
            pl.BlockSpec((EMB + 3, HID), lambda i: (0, 0)),
            pl.BlockSpec((1, HID), lambda i: (0, 0)),
            pl.BlockSpec((HID, HID), lambda i: (0, 0)),
            pl.BlockSpec((1, HID), lambda i: (0, 0)),
            pl.BlockSpec((HID, N_STRUCT), lambda i: (0, 0)),
            pl.BlockSpec((1, 1), lambda i: (0, 0)),
        ],
        out_specs=pl.BlockSpec((1, N_STRUCT), lambda i: (0, 0)),
        out_shape=jax.ShapeDtypeStruct((1, N_STRUCT), jnp.float32),
        scratch_shapes=[
            pltpu.VMEM((XDIM, HID), _BF16),
            pltpu.VMEM((XDIM, HID), _BF16),
        ],
    )(coord, elems3, ind3, elem_embed, W1, b1.reshape(1, HID), W2,
      b2.reshape(1, HID), w3t, b3.reshape(1, 1))
    return out[0]


# speedup vs baseline: 3.2248x; 1.2159x over previous
import jax
import jax.numpy as jnp
from jax.experimental import pallas as pl
from jax.experimental.pallas import tpu as pltpu

N_ATOMS = 16384
N_STRUCT = 16
N_ELEM = 95
EMB = 64
HID = 256
XDIM = 128

BLOCK = 8192
HALF = 2048

_F32 = jnp.float32
_BF16 = jnp.bfloat16


def _hi_lo(a):
    hi = a.astype(_BF16)
    lo = (a - hi.astype(_F32)).astype(_BF16)
    return hi, lo


def _fused_body(coord_ref, elems_ref, ind_ref, emb_ref, w1c_ref, b1_ref,
                w2_ref, b2_ref, w3_ref, b3_ref, out_ref, ws_ref):
    i = pl.program_id(0)

    @pl.when(i == 0)
    def _init():
        m = jnp.dot(emb_ref[...], w1c_ref[0:EMB, :],
                    preferred_element_type=_F32)
        mh, ml = _hi_lo(m)
        ws_ref[...] = jnp.concatenate([
            mh,
            w1c_ref[EMB:, :].astype(_BF16),
            b1_ref[...].astype(_BF16),
            jnp.zeros((XDIM - N_ELEM - 4, HID), _BF16),
            ml,
            jnp.zeros((XDIM - N_ELEM, HID), _BF16),
        ], axis=0)

    ws = ws_ref[...]
    b2 = b2_ref[0, :]
    w2 = w2_ref[...]
    w3t = w3_ref[...]
    b3 = b3_ref[0, 0]

    parts = []
    for h in range(BLOCK // HALF):
        sl = pl.ds(h * HALF, HALF)
        elems = elems_ref[0, 0, sl]
        onehot = (jax.lax.broadcasted_iota(jnp.int32, (HALF, N_ELEM), 1)
                  == elems[:, None]).astype(_F32)
        x = jnp.concatenate(
            [onehot, coord_ref[sl, :], jnp.ones((HALF, 1), _F32),
             jnp.zeros((HALF, XDIM - N_ELEM - 4), _F32)], axis=1)
        xb = x.astype(_BF16)
        x2 = jnp.concatenate([xb, xb], axis=1)
        h1 = jnp.tanh(jnp.dot(x2, ws, preferred_element_type=_F32))
        h2 = jnp.tanh(jnp.dot(h1, w2, preferred_element_type=_F32) + b2)
        pa16 = jnp.dot(h2, w3t, preferred_element_type=_F32) + b3
        ind = ind_ref[0, 0, sl]
        seg = (jax.lax.broadcasted_iota(jnp.int32, (HALF, N_STRUCT), 1)
               == ind[:, None]).astype(_F32)
        parts.append(jnp.sum(seg * pa16, axis=0)[None, :])
    part = parts[0]
    for p in parts[1:]:
        part = part + p

    @pl.when(i == 0)
    def _first():
        out_ref[...] = part

    @pl.when(i != 0)
    def _acc():
        out_ref[...] += part


@jax.jit
def kernel(coord, elems, ind_1, elem_embed, W1, b1, W2, b2, W3, b3):
    n = coord.shape[0]
    grid = n // BLOCK
    elems3 = elems.astype(jnp.int32).reshape(grid, 1, BLOCK)
    ind3 = ind_1.astype(jnp.int32).reshape(grid, 1, BLOCK)
    w3t = jnp.tile(W3, (1, N_STRUCT))

    out = pl.pallas_call(
        _fused_body,
        grid=(grid,),
        in_specs=[
            pl.BlockSpec((BLOCK, 3), lambda i: (i, 0)),
            pl.BlockSpec((1, 1, BLOCK), lambda i: (i, 0, 0)),
            pl.BlockSpec((1, 1, BLOCK), lambda i: (i, 0, 0)),
            pl.BlockSpec((N_ELEM, EMB), lambda i: (0, 0)),
            pl.BlockSpec((EMB + 3, HID), lambda i: (0, 0)),
            pl.BlockSpec((1, HID), lambda i: (0, 0)),
            pl.BlockSpec((HID, HID), lambda i: (0, 0)),
            pl.BlockSpec((1, HID), lambda i: (0, 0)),
            pl.BlockSpec((HID, N_STRUCT), lambda i: (0, 0)),
            pl.BlockSpec((1, 1), lambda i: (0, 0)),
        ],
        out_specs=pl.BlockSpec((1, N_STRUCT), lambda i: (0, 0)),
        out_shape=jax.ShapeDtypeStruct((1, N_STRUCT), jnp.float32),
        scratch_shapes=[
            pltpu.VMEM((2 * XDIM, HID), _BF16),
        ],
    )(coord, elems3, ind3, elem_embed, W1, b1.reshape(1, HID), W2,
      b2.reshape(1, HID), w3t, b3.reshape(1, 1))
    return out[0]
